# baseline (speedup 1.0000x reference)
import jax
import jax.numpy as jnp
from jax.experimental import pallas as pl

B, H, W_, CIN, COUT = 1024, 28, 28, 1, 32
BT = 32
KDIM = 91
NOUT = H * COUT


def _conv_body(m_ref, x_ref, out_ref):
    xb = x_ref[...]
    xp = jnp.pad(xb, ((0, 0), (1, 1), (1, 1)))
    r = BT * H
    slabs = [xp[:, dy:dy + H, :].reshape(r, W_ + 2) for dy in range(3)]
    slabs.append(jnp.ones((r, 1), dtype=jnp.float32))
    xall = jnp.concatenate(slabs, axis=1).astype(jnp.bfloat16)
    res = jnp.dot(xall, m_ref[...], preferred_element_type=jnp.float32)
    acc = res[:, :NOUT]
    xrep = res[:, NOUT:]
    out = jnp.where(xrep != 0.0, acc, 0.0)
    out_ref[...] = out.reshape(r * 7, 128)


@jax.jit
def kernel(x, conv_w, conv_b):
    xsq = x.reshape(B, H, W_)

    sel = jnp.stack([jnp.eye(W_, W_ + 2, k=dx, dtype=jnp.float32)
                     for dx in range(3)])
    m = jnp.einsum("xwj,yxc->ywjc", sel, conv_w[:, :, 0, :])
    m = m.transpose(0, 2, 1, 3).reshape(3 * (W_ + 2), NOUT)
    bias_row = jnp.tile(conv_b, W_).reshape(1, NOUT)
    m = jnp.concatenate([m, bias_row], axis=0)
    e = jnp.repeat(sel[1], COUT, axis=0).T
    e = jnp.concatenate([jnp.zeros((30, NOUT)), e,
                         jnp.zeros((31, NOUT))], axis=0)
    mfull = jnp.concatenate([m, e], axis=1).astype(jnp.bfloat16)

    grid = B // BT
    out = pl.pallas_call(
        _conv_body,
        grid=(grid,),
        in_specs=[
            pl.BlockSpec((KDIM, 2 * NOUT), lambda i: (0, 0)),
            pl.BlockSpec((BT, H, W_), lambda i: (i, 0, 0)),
        ],
        out_specs=pl.BlockSpec((BT * H * 7, 128), lambda i: (i, 0)),
        out_shape=jax.ShapeDtypeStruct((B * H * 7, 128), jnp.float32),
    )(mfull, xsq)
    return out.reshape(B * H * W_, COUT)


# speedup vs baseline: 2.1753x; 2.1753x over previous
import jax
import jax.numpy as jnp
import numpy as np
from jax.experimental import pallas as pl

B, H, W_, CIN, COUT = 1024, 28, 28, 1, 32
NPIX = B * H * W_
GRID = 16
PB = NPIX // GRID
MROW = 16 * H * W_
_SHIFTS = [W_ * dy + dx for dy in (-1, 0, 1) for dx in (-1, 0, 1)]


def _shift_lanes(row, s):
    if s == 0:
        return row
    z = jnp.zeros((1, abs(s)), dtype=row.dtype)
    if s > 0:
        return jnp.concatenate([row[:, s:], z], axis=1)
    return jnp.concatenate([z, row[:, :s]], axis=1)


def _conv_body(w_ref, b_ref, m_ref, x_ref, out_ref):
    xrow = x_ref[...].reshape(1, PB)
    m9 = jnp.concatenate([m_ref[...]] * (PB // MROW), axis=1)
    acc = jnp.broadcast_to(b_ref[...], (COUT, PB))
    for k, s in enumerate(_SHIFTS):
        sk = _shift_lanes(xrow, s) * m9[k:k + 1, :]
        acc = acc + sk * w_ref[:, k:k + 1]
    out_ref[...] = jnp.where(xrow != 0.0, acc, 0.0)


def _tap_masks():
    q = np.arange(MROW) % (H * W_)
    h, w = q // W_, q % W_
    rows = []
    for dy in (-1, 0, 1):
        for dx in (-1, 0, 1):
            rows.append(((h + dy >= 0) & (h + dy < H)
                         & (w + dx >= 0) & (w + dx < W_)))
    return jnp.asarray(np.stack(rows).astype(np.float32))


_MASKS = _tap_masks()


@jax.jit
def kernel(x, conv_w, conv_b):
    x5 = x.reshape(GRID, 1, PB)
    wt = conv_w.reshape(9, COUT).T
    bcol = conv_b.reshape(COUT, 1)

    out_t = pl.pallas_call(
        _conv_body,
        grid=(GRID,),
        in_specs=[
            pl.BlockSpec((COUT, 9), lambda i: (0, 0)),
            pl.BlockSpec((COUT, 1), lambda i: (0, 0)),
            pl.BlockSpec((9, MROW), lambda i: (0, 0)),
            pl.BlockSpec((1, 1, PB), lambda i: (i, 0, 0)),
        ],
        out_specs=pl.BlockSpec((COUT, PB), lambda i: (0, i)),
        out_shape=jax.ShapeDtypeStruct((COUT, NPIX), jnp.float32),
    )(wt, bcol, _MASKS, x5)
    return out_t.T


# speedup vs baseline: 5.0965x; 2.3429x over previous
import jax
import jax.numpy as jnp
import numpy as np
from jax.experimental import pallas as pl

B, H, W_, CIN, COUT = 1024, 28, 28, 1, 32
NPIX = B * H * W_
GRID = 16
PB = NPIX // GRID
MROW = 16 * H * W_
_SHIFTS = [W_ * dy + dx for dy in (-1, 0, 1) for dx in (-1, 0, 1)]


def _shift_lanes(row, s):
    if s == 0:
        return row
    z = jnp.zeros((1, abs(s)), dtype=row.dtype)
    if s > 0:
        return jnp.concatenate([row[:, s:], z], axis=1)
    return jnp.concatenate([z, row[:, :s]], axis=1)


def _conv_body(w_ref, m_ref, x_ref, out_ref):
    xrow = x_ref[...].reshape(1, PB)
    m9 = jnp.concatenate([m_ref[...]] * (PB // MROW), axis=1)
    rows = [_shift_lanes(xrow, s) * m9[k:k + 1, :]
            for k, s in enumerate(_SHIFTS)]
    rows.append(jnp.ones((1, PB), dtype=jnp.float32))
    taps = jnp.concatenate(rows, axis=0)
    acc = jnp.dot(w_ref[...], taps,
                  preferred_element_type=jnp.float32)
    out_ref[...] = jnp.where(xrow != 0.0, acc, 0.0)


def _tap_masks():
    q = np.arange(MROW) % (H * W_)
    h, w = q // W_, q % W_
    rows = []
    for dy in (-1, 0, 1):
        for dx in (-1, 0, 1):
            rows.append(((h + dy >= 0) & (h + dy < H)
                         & (w + dx >= 0) & (w + dx < W_)))
    return np.stack(rows).astype(np.float32)


_MASKS = _tap_masks()


@jax.jit
def kernel(x, conv_w, conv_b):
    x5 = x.reshape(GRID, 1, PB)
    wt = jnp.concatenate(
        [conv_w.reshape(9, COUT).T, conv_b.reshape(COUT, 1)],
        axis=1)

    out_t = pl.pallas_call(
        _conv_body,
        grid=(GRID,),
        in_specs=[
            pl.BlockSpec((COUT, 10), lambda i: (0, 0)),
            pl.BlockSpec((9, MROW), lambda i: (0, 0)),
            pl.BlockSpec((1, 1, PB), lambda i: (i, 0, 0)),
        ],
        out_specs=pl.BlockSpec((COUT, PB), lambda i: (0, i)),
        out_shape=jax.ShapeDtypeStruct((COUT, NPIX), jnp.float32),
    )(wt, jnp.asarray(_MASKS), x5)
    return out_t.T


# speedup vs baseline: 5.5115x; 1.0814x over previous
import jax
import jax.numpy as jnp
import numpy as np
from jax.experimental import pallas as pl
from jax.experimental.pallas import tpu as pltpu

B, H, W_, CIN, COUT = 1024, 28, 28, 1, 32
NPIX = B * H * W_
GRID = 8
PB = NPIX // GRID
MROW = 16 * H * W_
_SHIFTS = [W_ * dy + dx for dy in (-1, 0, 1) for dx in (-1, 0, 1)]


def _shift_lanes(row, s):
    if s == 0:
        return row
    z = jnp.zeros((1, abs(s)), dtype=row.dtype)
    if s > 0:
        return jnp.concatenate([row[:, s:], z], axis=1)
    return jnp.concatenate([z, row[:, :s]], axis=1)


def _conv_body(w_ref, m_ref, x_ref, out_ref, xs_ref):
    xt = x_ref[...].reshape(H * W_, B // GRID).T
    for b in range(B // GRID):
        xs_ref[:, b * H * W_:(b + 1) * H * W_] = xt[b:b + 1, :]
    xrow = xs_ref[...]
    m9 = jnp.concatenate([m_ref[...]] * (PB // MROW), axis=1)
    rows = [_shift_lanes(xrow, s) * m9[k:k + 1, :]
            for k, s in enumerate(_SHIFTS)]
    rows.append(jnp.ones((1, PB), dtype=jnp.float32))
    taps = jnp.concatenate(rows, axis=0)
    acc = jnp.dot(w_ref[...], taps,
                  preferred_element_type=jnp.float32)
    out_ref[...] = jnp.where(xrow != 0.0, acc, 0.0)


def _tap_masks():
    q = np.arange(MROW) % (H * W_)
    h, w = q // W_, q % W_
    rows = []
    for dy in (-1, 0, 1):
        for dx in (-1, 0, 1):
            rows.append(((h + dy >= 0) & (h + dy < H)
                         & (w + dx >= 0) & (w + dx < W_)))
    return np.stack(rows).astype(np.float32)


_MASKS = _tap_masks()


@jax.jit
def kernel(x, conv_w, conv_b):
    x5 = x.reshape(B, H * W_).T.reshape(H * W_, 1, B)
    wt = jnp.concatenate(
        [conv_w.reshape(9, COUT).T, conv_b.reshape(COUT, 1)],
        axis=1)

    out_t = pl.pallas_call(
        _conv_body,
        grid=(GRID,),
        in_specs=[
            pl.BlockSpec((COUT, 10), lambda i: (0, 0)),
            pl.BlockSpec((9, MROW), lambda i: (0, 0)),
            pl.BlockSpec((H * W_, 1, B // GRID), lambda i: (0, 0, i)),
        ],
        out_specs=pl.BlockSpec((COUT, PB), lambda i: (0, i)),
        out_shape=jax.ShapeDtypeStruct((COUT, NPIX), jnp.float32),
        scratch_shapes=[pltpu.VMEM((1, PB), jnp.float32)],
    )(wt, jnp.asarray(_MASKS), x5)
    return out_t.T
